# baseline (speedup 1.0000x reference)
import functools

import jax
import jax.numpy as jnp
from jax import lax
from jax.experimental import pallas as pl
from jax.experimental.pallas import tpu as pltpu
from jax.experimental.pallas import tpu_sc as plsc

_GAMMA = 0.1
_EPSILON = 0.1
_NC = 2
_NS = 16
_CH = 80
_NP = 10240
_BR = 1000


@functools.lru_cache(maxsize=None)
def _make_deg_kernel(e):
    nw = _NC * _NS
    ep = e // nw
    cpt = ep // _CH
    mesh = plsc.VectorSubcoreMesh(core_axis_name="c", subcore_axis_name="s")

    @functools.partial(
        pl.kernel,
        out_type=jax.ShapeDtypeStruct((_NC, 1, _NP), jnp.float32),
        mesh=mesh,
        scratch_types=[
            pltpu.VMEM((cpt, _CH), jnp.int32),
            pltpu.VMEM((_CH,), jnp.float32),
            pltpu.VMEM_SHARED((_NP,), jnp.float32),
            pltpu.SemaphoreType.DMA,
        ],
    )
    def deg_kernel(dst3d_hbm, zeros_hbm, degp_hbm, idx_v, ones_v, shared_deg,
                   dsem):
        c = lax.axis_index("c")
        s = lax.axis_index("s")
        wid = c * _NS + s

        @pl.when(s == 0)
        def _():
            pltpu.sync_copy(zeros_hbm, shared_deg)

        pltpu.sync_copy(dst3d_hbm.at[wid], idx_v)
        for i in range(_CH // 16):
            ones_v[pl.ds(i * 16, 16)] = jnp.ones((16,), jnp.float32)
        plsc.subcore_barrier()

        def body(j, carry):
            pltpu.async_copy(ones_v, shared_deg.at[idx_v.at[j]], dsem,
                             add=True)

            @pl.when(j >= 8)
            def _():
                pltpu.make_async_copy(ones_v, shared_deg.at[idx_v.at[j]],
                                      dsem).wait()
            return carry

        lax.fori_loop(0, cpt, body, 0)

        def drain(j, carry):
            pltpu.make_async_copy(ones_v, shared_deg.at[idx_v.at[0]],
                                  dsem).wait()
            return carry

        lax.fori_loop(0, min(8, cpt), drain, 0)
        plsc.subcore_barrier()

        @pl.when(s == 0)
        def _():
            pltpu.sync_copy(shared_deg, degp_hbm.at[c, 0])

    return deg_kernel


_KB = 5
_CHA = 40


@functools.lru_cache(maxsize=None)
def _make_agg_kernel(e, d):
    nw = _NC * _NS
    ep = e // nw
    cpt = ep // _CHA
    assert cpt % (2 * _KB) == 0
    grp = cpt // _KB
    rpt = _NP // _NS
    mesh = plsc.VectorSubcoreMesh(core_axis_name="c", subcore_axis_name="s")

    @functools.partial(
        pl.kernel,
        out_type=jax.ShapeDtypeStruct((_NC, _NP, d), jnp.float32),
        mesh=mesh,
        scratch_types=[
            pltpu.VMEM((2, _KB, _CHA), jnp.int32),
            pltpu.VMEM((2, _KB, _CHA), jnp.int32),
        ] + [pltpu.VMEM((_CHA, d), jnp.float32) for _ in range(_KB)] + [
            pltpu.VMEM_SHARED((_NP, d), jnp.float32),
            pltpu.SemaphoreType.DMA,
            pltpu.SemaphoreType.DMA,
            pltpu.SemaphoreType.DMA,
        ] + [pltpu.SemaphoreType.DMA for _ in range(_KB)],
    )
    def agg_kernel(idx5d, g_hbm, znd_hbm, parts_hbm,
                   idx0, idx1, *rest):
        rows = rest[:_KB]
        shared_agg = rest[_KB]
        isem0, isem1 = rest[_KB + 1], rest[_KB + 2]
        ssem = rest[_KB + 3]
        gsems = rest[_KB + 4:]
        c = lax.axis_index("c")
        s = lax.axis_index("s")
        wid = c * _NS + s

        pltpu.sync_copy(znd_hbm.at[pl.ds(s * rpt, rpt)],
                        shared_agg.at[pl.ds(s * rpt, rpt)])
        pltpu.async_copy(idx5d.at[wid, 0], idx0, isem0)
        plsc.subcore_barrier()

        def body(jj2, carry):
            for b2, ibuf, isem, iother, iosem in (
                    (0, idx0, isem0, idx1, isem1),
                    (1, idx1, isem1, idx0, isem0)):
                jj = jj2 * 2 + b2
                pltpu.make_async_copy(idx5d.at[wid, jj], ibuf, isem).wait()

                @pl.when(jj >= 1)
                def _():
                    for b in range(_KB):
                        pltpu.make_async_copy(
                            rows[b], shared_agg.at[ibuf.at[1, b]],
                            ssem).wait()

                @pl.when(jj + 1 < grp)
                def _():
                    pltpu.async_copy(idx5d.at[wid, jj + 1], iother, iosem)

                handles = [
                    pltpu.async_copy(g_hbm.at[ibuf.at[0, b]],
                                     rows[b], gsems[b])
                    for b in range(_KB)
                ]
                for b in range(_KB):
                    handles[b].wait()
                    pltpu.async_copy(rows[b], shared_agg.at[ibuf.at[1, b]],
                                     ssem, add=True)
            return carry

        lax.fori_loop(0, grp // 2, body, 0)
        for b in range(_KB):
            pltpu.make_async_copy(rows[b], shared_agg.at[idx1.at[1, b]],
                                  ssem).wait()
        plsc.subcore_barrier()

        pltpu.sync_copy(shared_agg.at[pl.ds(s * rpt, rpt)],
                        parts_hbm.at[c, pl.ds(s * rpt, rpt)])

    return agg_kernel


def _tc1_body(x_ref, wphi_ref, w_ref, h_ref, z_ref):
    x = x_ref[...]
    w = w_ref[...]
    h_ref[...] = jnp.dot(x, wphi_ref[...],
                         preferred_element_type=jnp.float32,
                         precision=lax.Precision.HIGHEST)
    a_t = w.T - w
    z_ref[...] = jnp.dot(x, a_t,
                         preferred_element_type=jnp.float32,
                         precision=lax.Precision.HIGHEST) - _GAMMA * x


def _tc2_body(degt_ref, h_ref, g_ref):
    dp = degt_ref[...]
    dis = lax.rsqrt(dp[:, 0:1] + dp[:, 1:2] + 1.0)
    g_ref[...] = dis * h_ref[...]


def _tc3_body(x_ref, z_ref, g_ref, parts_ref, degt_ref, bias_ref, out_ref):
    dp = degt_ref[...]
    dis = lax.rsqrt(dp[:, 0:1] + dp[:, 1:2] + 1.0)
    agg = parts_ref[0] + parts_ref[1] + g_ref[...]
    pre = z_ref[...] + dis * agg + bias_ref[...]
    out_ref[...] = x_ref[...] + _EPSILON * jnp.tanh(pre)


def _tc1(x, w_phi, w):
    n, d = x.shape
    grid = (n // _BR,)
    row = pl.BlockSpec((_BR, d), lambda i: (i, 0))
    full = pl.BlockSpec((d, d), lambda i: (0, 0))
    return pl.pallas_call(
        _tc1_body,
        grid=grid,
        in_specs=[row, full, full],
        out_specs=[row, row],
        out_shape=[jax.ShapeDtypeStruct((n, d), jnp.float32)] * 2,
    )(x, w_phi, w)


def _tc2(degt, h):
    n, d = h.shape
    grid = (n // _BR,)
    row = pl.BlockSpec((_BR, d), lambda i: (i, 0))
    degs = pl.BlockSpec((_BR, _NC), lambda i: (i, 0))
    return pl.pallas_call(
        _tc2_body,
        grid=grid,
        in_specs=[degs, row],
        out_specs=row,
        out_shape=jax.ShapeDtypeStruct((n, d), jnp.float32),
    )(degt, h)


def _tc3(x, z, g, parts, degt, bias2d):
    n, d = x.shape
    grid = (n // _BR,)
    row = pl.BlockSpec((_BR, d), lambda i: (i, 0))
    pspec = pl.BlockSpec((_NC, _BR, d), lambda i: (0, i, 0))
    degs = pl.BlockSpec((_BR, _NC), lambda i: (i, 0))
    bspec = pl.BlockSpec((1, d), lambda i: (0, 0))
    return pl.pallas_call(
        _tc3_body,
        grid=grid,
        in_specs=[row, row, row, pspec, degs, bspec],
        out_specs=row,
        out_shape=jax.ShapeDtypeStruct((n, d), jnp.float32),
    )(x, z, g, parts, degt, bias2d)


def kernel(x, edge_index, W, W_phi, bias):
    n, d = x.shape
    e = edge_index.shape[1]
    nw = _NC * _NS
    cpt = e // (nw * _CH)

    grp = e // (nw * _CHA * _KB)

    dst3d = edge_index[1].reshape(nw, cpt, _CH)
    src_r = edge_index[0].reshape(nw, grp, 1, _KB, _CHA)
    dst_r = edge_index[1].reshape(nw, grp, 1, _KB, _CHA)
    idx5d = jnp.concatenate([src_r, dst_r], axis=2)
    zeros_n = jnp.zeros((_NP,), jnp.float32)
    zeros_nd = jnp.zeros((_NP, d), jnp.float32)

    degp = _make_deg_kernel(e)(dst3d, zeros_n)
    h, z = _tc1(x, W_phi, W)
    degt = degp.reshape(_NC, _NP)[:, :n].T
    g = _tc2(degt, h)
    parts = _make_agg_kernel(e, d)(idx5d, g, zeros_nd)
    out = _tc3(x, z, g, parts, degt, bias.reshape(1, d))
    return out


# speedup vs baseline: 1.0442x; 1.0442x over previous
import functools

import jax
import jax.numpy as jnp
from jax import lax
from jax.experimental import pallas as pl
from jax.experimental.pallas import tpu as pltpu
from jax.experimental.pallas import tpu_sc as plsc

_GAMMA = 0.1
_EPSILON = 0.1
_NC = 2
_NS = 16
_CH = 80
_NP = 10240
_BR = 1000


@functools.lru_cache(maxsize=None)
def _make_deg_kernel(e):
    nw = _NC * _NS
    ep = e // nw
    cpt = ep // _CH
    mesh = plsc.VectorSubcoreMesh(core_axis_name="c", subcore_axis_name="s")

    @functools.partial(
        pl.kernel,
        out_type=jax.ShapeDtypeStruct((_NC, 1, _NP), jnp.float32),
        mesh=mesh,
        scratch_types=[
            pltpu.VMEM((cpt, _CH), jnp.int32),
            pltpu.VMEM((_CH,), jnp.float32),
            pltpu.VMEM_SHARED((_NP,), jnp.float32),
            pltpu.SemaphoreType.DMA,
        ],
    )
    def deg_kernel(dst3d_hbm, zeros_hbm, degp_hbm, idx_v, ones_v, shared_deg,
                   dsem):
        c = lax.axis_index("c")
        s = lax.axis_index("s")
        wid = c * _NS + s

        @pl.when(s == 0)
        def _():
            pltpu.sync_copy(zeros_hbm, shared_deg)

        pltpu.sync_copy(dst3d_hbm.at[wid], idx_v)
        for i in range(_CH // 16):
            ones_v[pl.ds(i * 16, 16)] = jnp.ones((16,), jnp.float32)
        plsc.subcore_barrier()

        def body(j, carry):
            pltpu.async_copy(ones_v, shared_deg.at[idx_v.at[j]], dsem,
                             add=True)

            @pl.when(j >= 8)
            def _():
                pltpu.make_async_copy(ones_v, shared_deg.at[idx_v.at[j]],
                                      dsem).wait()
            return carry

        lax.fori_loop(0, cpt, body, 0)

        def drain(j, carry):
            pltpu.make_async_copy(ones_v, shared_deg.at[idx_v.at[0]],
                                  dsem).wait()
            return carry

        lax.fori_loop(0, min(8, cpt), drain, 0)
        plsc.subcore_barrier()

        @pl.when(s == 0)
        def _():
            pltpu.sync_copy(shared_deg, degp_hbm.at[c, 0])

    return deg_kernel


_KB = 5
_CHA = 40


@functools.lru_cache(maxsize=None)
def _make_agg_kernel(e, d):
    nw = _NC * _NS
    ep = e // nw
    cpt = ep // _CHA
    assert cpt % (2 * _KB) == 0
    grp = cpt // _KB
    rpt = _NP // _NS
    mesh = plsc.VectorSubcoreMesh(core_axis_name="c", subcore_axis_name="s")

    @functools.partial(
        pl.kernel,
        out_type=jax.ShapeDtypeStruct((_NC, _NP, d), jnp.float32),
        mesh=mesh,
        scratch_types=[
            pltpu.VMEM((2, _KB, _CHA), jnp.int32),
            pltpu.VMEM((2, _KB, _CHA), jnp.int32),
        ] + [pltpu.VMEM((_CHA, d), jnp.float32) for _ in range(_KB)] + [
            pltpu.VMEM_SHARED((_NP, d), jnp.float32),
            pltpu.SemaphoreType.DMA,
            pltpu.SemaphoreType.DMA,
            pltpu.SemaphoreType.DMA,
        ] + [pltpu.SemaphoreType.DMA for _ in range(_KB)],
    )
    def agg_kernel(idx5d, g_hbm, znd_hbm, parts_hbm,
                   idx0, idx1, *rest):
        rows = rest[:_KB]
        shared_agg = rest[_KB]
        isem0, isem1 = rest[_KB + 1], rest[_KB + 2]
        ssem = rest[_KB + 3]
        gsems = rest[_KB + 4:]
        c = lax.axis_index("c")
        s = lax.axis_index("s")
        wid = c * _NS + s

        pltpu.sync_copy(znd_hbm.at[pl.ds(s * rpt, rpt)],
                        shared_agg.at[pl.ds(s * rpt, rpt)])
        pltpu.async_copy(idx5d.at[wid, 0], idx0, isem0)
        plsc.subcore_barrier()

        def body(jj2, carry):
            for b2, ibuf, isem, iother, iosem in (
                    (0, idx0, isem0, idx1, isem1),
                    (1, idx1, isem1, idx0, isem0)):
                jj = jj2 * 2 + b2
                pltpu.make_async_copy(idx5d.at[wid, jj], ibuf, isem).wait()

                @pl.when(jj >= 1)
                def _():
                    for b in range(_KB):
                        pltpu.make_async_copy(
                            rows[b], shared_agg.at[ibuf.at[1, b]],
                            ssem).wait()

                @pl.when(jj + 1 < grp)
                def _():
                    pltpu.async_copy(idx5d.at[wid, jj + 1], iother, iosem)

                handles = [
                    pltpu.async_copy(g_hbm.at[ibuf.at[0, b]],
                                     rows[b], gsems[b])
                    for b in range(_KB)
                ]
                for b in range(_KB):
                    handles[b].wait()
                    pltpu.async_copy(rows[b], shared_agg.at[ibuf.at[1, b]],
                                     ssem, add=True)
            return carry

        lax.fori_loop(0, grp // 2, body, 0)
        for b in range(_KB):
            pltpu.make_async_copy(rows[b], shared_agg.at[idx1.at[1, b]],
                                  ssem).wait()
        plsc.subcore_barrier()

        pltpu.sync_copy(shared_agg.at[pl.ds(s * rpt, rpt)],
                        parts_hbm.at[c, pl.ds(s * rpt, rpt)])

    return agg_kernel


def _tc1_body(degt_ref, x_ref, wphi_ref, w_ref, g_ref, z_ref):
    x = x_ref[...]
    w = w_ref[...]
    dp = degt_ref[...]
    dis = lax.rsqrt(dp[:, 0:1] + dp[:, 1:2] + 1.0)
    g_ref[...] = dis * jnp.dot(x, wphi_ref[...],
                               preferred_element_type=jnp.float32,
                               precision=lax.Precision.HIGHEST)
    a_t = w.T - w
    z_ref[...] = jnp.dot(x, a_t,
                         preferred_element_type=jnp.float32,
                         precision=lax.Precision.HIGHEST) - _GAMMA * x


def _tc3_body(x_ref, z_ref, g_ref, parts_ref, degt_ref, bias_ref, out_ref):
    dp = degt_ref[...]
    dis = lax.rsqrt(dp[:, 0:1] + dp[:, 1:2] + 1.0)
    agg = parts_ref[0] + parts_ref[1] + g_ref[...]
    pre = z_ref[...] + dis * agg + bias_ref[...]
    out_ref[...] = x_ref[...] + _EPSILON * jnp.tanh(pre)


def _tc1(degt, x, w_phi, w):
    n, d = x.shape
    grid = (n // _BR,)
    row = pl.BlockSpec((_BR, d), lambda i: (i, 0))
    full = pl.BlockSpec((d, d), lambda i: (0, 0))
    degs = pl.BlockSpec((_BR, _NC), lambda i: (i, 0))
    return pl.pallas_call(
        _tc1_body,
        grid=grid,
        in_specs=[degs, row, full, full],
        out_specs=[row, row],
        out_shape=[jax.ShapeDtypeStruct((n, d), jnp.float32)] * 2,
    )(degt, x, w_phi, w)


def _tc3(x, z, g, parts, degt, bias2d):
    n, d = x.shape
    grid = (n // _BR,)
    row = pl.BlockSpec((_BR, d), lambda i: (i, 0))
    pspec = pl.BlockSpec((_NC, _BR, d), lambda i: (0, i, 0))
    degs = pl.BlockSpec((_BR, _NC), lambda i: (i, 0))
    bspec = pl.BlockSpec((1, d), lambda i: (0, 0))
    return pl.pallas_call(
        _tc3_body,
        grid=grid,
        in_specs=[row, row, row, pspec, degs, bspec],
        out_specs=row,
        out_shape=jax.ShapeDtypeStruct((n, d), jnp.float32),
    )(x, z, g, parts, degt, bias2d)


def kernel(x, edge_index, W, W_phi, bias):
    n, d = x.shape
    e = edge_index.shape[1]
    nw = _NC * _NS
    cpt = e // (nw * _CH)

    grp = e // (nw * _CHA * _KB)

    dst3d = edge_index[1].reshape(nw, cpt, _CH)
    src_r = edge_index[0].reshape(nw, grp, 1, _KB, _CHA)
    dst_r = edge_index[1].reshape(nw, grp, 1, _KB, _CHA)
    idx5d = jnp.concatenate([src_r, dst_r], axis=2)
    zeros_n = jnp.zeros((_NP,), jnp.float32)
    zeros_nd = jnp.zeros((_NP, d), jnp.float32)

    degp = _make_deg_kernel(e)(dst3d, zeros_n)
    degt = degp.reshape(_NC, _NP)[:, :n].T
    g, z = _tc1(degt, x, W_phi, W)
    parts = _make_agg_kernel(e, d)(idx5d, g, zeros_nd)
    out = _tc3(x, z, g, parts, degt, bias.reshape(1, d))
    return out
